# baseline (speedup 1.0000x reference)
from math import comb

import jax
import jax.numpy as jnp
import numpy as np
from jax import lax
from jax.experimental import pallas as pl
from jax.experimental.pallas import tpu as pltpu

_H, _W = 224, 224
_S = 15
_K = 4
_EPS = 1e-06
_R, _C = 392, 128


def _basis() -> jnp.ndarray:
    ts = jnp.linspace(0.0, 1.0, _S)
    i = np.arange(_K)
    coeff = np.array([comb(_K - 1, j) for j in range(_K)], dtype=np.float32)
    return (coeff[None, :] * (ts[:, None] ** i[None, :])
            * ((1.0 - ts[:, None]) ** (_K - 1 - i[None, :]))).astype(jnp.float32)


def _curve_kernel(s2_ref, ym_ref, xm_ref, w_ref, wr_ref, aa_ref, out_ref,
                  yf_s, xf_s, p2_s):
    b = pl.program_id(0)

    @pl.when(b == 0)
    def _init():
        ri = lax.broadcasted_iota(jnp.int32, (_R, _C), 0)
        ci = lax.broadcasted_iota(jnp.int32, (_R, _C), 1)
        nf = (ri * _C + ci).astype(jnp.float32)
        yf = jnp.floor((nf + 0.5) * (1.0 / 224.0))
        xf = nf - yf * 224.0
        yf_s[...] = yf
        xf_s[...] = xf
        p2_s[...] = yf * yf + xf * xf

    yf = yf_s[...]
    xf = xf_s[...]
    p2 = p2_s[...]

    m = None
    for s in range(_S):
        v = yf * ym_ref[b, s] + xf * xm_ref[b, s]
        d2 = (p2 + v) + s2_ref[b, s]
        m = d2 if m is None else jnp.minimum(m, d2)

    dmin = jnp.sqrt(jnp.maximum(m, 0.0) + 1e-12)
    base = dmin * wr_ref[b] + _EPS
    val = 1.0 - jnp.exp(aa_ref[b] * jnp.log(base))
    out_ref[0] = jnp.where(dmin < w_ref[b], val, 0.0)


@jax.jit
def kernel(inputs, widths, aa_factors):
    B = inputs.shape[0]
    kp = inputs * jnp.array([float(_H), float(_W)], dtype=jnp.float32)
    sp = jnp.einsum('sk,bkd->bsd', _basis(), kp)
    s2 = jnp.sum(sp * sp, axis=-1)

    def _rne_bf16(x):
        u = lax.bitcast_convert_type(x, jnp.uint32)
        u = u + jnp.uint32(0x7FFF) + ((u >> 16) & jnp.uint32(1))
        return lax.bitcast_convert_type(u & jnp.uint32(0xFFFF0000), jnp.float32)

    ym = -2.0 * _rne_bf16(sp[:, :, 0])
    xm = -2.0 * _rne_bf16(sp[:, :, 1])
    wr = 1.0 / widths

    out = pl.pallas_call(
        _curve_kernel,
        grid=(B,),
        in_specs=[pl.BlockSpec(memory_space=pltpu.SMEM)] * 6,
        out_specs=pl.BlockSpec((1, _R, _C), lambda b: (b, 0, 0)),
        out_shape=jax.ShapeDtypeStruct((B, _R, _C), jnp.float32),
        scratch_shapes=[pltpu.VMEM((_R, _C), jnp.float32)] * 3,
    )(s2, ym, xm, widths, wr, aa_factors)
    return out.reshape(B, _H, _W)


# speedup vs baseline: 1.1792x; 1.1792x over previous
from math import comb

import jax
import jax.numpy as jnp
import numpy as np
from jax import lax
from jax.experimental import pallas as pl
from jax.experimental.pallas import tpu as pltpu

_H, _W = 224, 224
_S = 15
_K = 4
_EPS = 1e-06


def _basis() -> jnp.ndarray:
    ts = jnp.linspace(0.0, 1.0, _S)
    i = np.arange(_K)
    coeff = np.array([comb(_K - 1, j) for j in range(_K)], dtype=np.float32)
    return (coeff[None, :] * (ts[:, None] ** i[None, :])
            * ((1.0 - ts[:, None]) ** (_K - 1 - i[None, :]))).astype(jnp.float32)


def _curve_kernel(s2_ref, ym_ref, xm_ref, w_ref, wr_ref, aa_ref, out_ref,
                  yf_s, xf_s, p2_s):
    b = pl.program_id(0)

    @pl.when(b == 0)
    def _init():
        yf = lax.broadcasted_iota(jnp.int32, (_H, _W), 0).astype(jnp.float32)
        xf = lax.broadcasted_iota(jnp.int32, (_H, _W), 1).astype(jnp.float32)
        yf_s[...] = yf
        xf_s[...] = xf
        p2_s[...] = yf * yf + xf * xf

    yf = yf_s[...]
    xf = xf_s[...]
    p2 = p2_s[...]

    m = None
    for s in range(_S):
        v = yf * ym_ref[b, s] + xf * xm_ref[b, s]
        d2 = (p2 + v) + s2_ref[b, s]
        m = d2 if m is None else jnp.minimum(m, d2)

    dmin = jnp.sqrt(jnp.maximum(m, 0.0) + 1e-12)
    base = dmin * wr_ref[b] + _EPS
    val = 1.0 - jnp.exp(aa_ref[b] * jnp.log(base))
    out_ref[0] = jnp.where(dmin < w_ref[b], val, 0.0)


@jax.jit
def kernel(inputs, widths, aa_factors):
    B = inputs.shape[0]
    kp = inputs * jnp.array([float(_H), float(_W)], dtype=jnp.float32)
    sp = jnp.einsum('sk,bkd->bsd', _basis(), kp)
    s2 = jnp.sum(sp * sp, axis=-1)

    def _rne_bf16(x):
        u = lax.bitcast_convert_type(x, jnp.uint32)
        u = u + jnp.uint32(0x7FFF) + ((u >> 16) & jnp.uint32(1))
        return lax.bitcast_convert_type(u & jnp.uint32(0xFFFF0000), jnp.float32)

    ym = -2.0 * _rne_bf16(sp[:, :, 0])
    xm = -2.0 * _rne_bf16(sp[:, :, 1])
    wr = 1.0 / widths

    return pl.pallas_call(
        _curve_kernel,
        grid=(B,),
        in_specs=[pl.BlockSpec(memory_space=pltpu.SMEM)] * 6,
        out_specs=pl.BlockSpec((1, _H, _W), lambda b: (b, 0, 0)),
        out_shape=jax.ShapeDtypeStruct((B, _H, _W), jnp.float32),
        scratch_shapes=[pltpu.VMEM((_H, _W), jnp.float32)] * 3,
    )(s2, ym, xm, widths, wr, aa_factors)
